# Initial kernel scaffold; baseline (speedup 1.0000x reference)
import jax
import jax.numpy as jnp
from jax.experimental import pallas as pl


def kernel(x, edge_index, W1, b1, W2, b2):
    raise NotImplementedError("write your pallas kernel here")


# speedup vs baseline: 16.6197x; 16.6197x over previous
import functools

import jax
import jax.numpy as jnp
from jax import lax
from jax.experimental import pallas as pl
from jax.experimental.pallas import tpu as pltpu
from jax.experimental.pallas import tpu_sc as plsc

N = 10000
N_PAD = 10240
F = 128
E_PAD = 327680
EROWS = E_PAD // 128
NC, NS = 2, 16
ROWS_PER_TILE = N_PAD // NS


def _mesh():
    return plsc.VectorSubcoreMesh(core_axis_name="c", subcore_axis_name="s",
                                  num_cores=NC, num_subcores=NS)


def _deg_body(dst_hbm, ones_hbm, zeros_hbm, deg_out, dstv, onesv, zerosv,
              deg_sh, sem):
    c = lax.axis_index("c")
    s = lax.axis_index("s")
    nrows = EROWS // (NC * NS)
    row0 = (c * NS + s) * nrows

    pltpu.sync_copy(zeros_hbm.at[pl.ds(0, ROWS_PER_TILE)], zerosv)
    pltpu.sync_copy(zerosv, deg_sh.at[pl.ds(s * ROWS_PER_TILE, ROWS_PER_TILE)])
    pltpu.sync_copy(ones_hbm.at[pl.ds(0, 128)], onesv)
    pltpu.sync_copy(dst_hbm.at[pl.ds(row0, nrows)], dstv)
    plsc.subcore_barrier()

    def step(g, carry):
        pltpu.sync_copy(onesv, deg_sh.at[dstv.at[g]], add=True)
        return carry

    lax.fori_loop(0, nrows, step, 0, unroll=False)
    plsc.subcore_barrier()
    pltpu.sync_copy(deg_sh.at[pl.ds(s * ROWS_PER_TILE, ROWS_PER_TILE)],
                    deg_out.at[c, pl.ds(s * ROWS_PER_TILE, ROWS_PER_TILE)])


def _deg_kernel(dst2d, ones_hbm, zeros_hbm):
    return pl.kernel(
        _deg_body,
        out_type=jax.ShapeDtypeStruct((NC, N_PAD), jnp.float32),
        mesh=_mesh(),
        scratch_types=[
            pltpu.VMEM((EROWS // (NC * NS), 128), jnp.int32),
            pltpu.VMEM((128,), jnp.float32),
            pltpu.VMEM((ROWS_PER_TILE,), jnp.float32),
            pltpu.VMEM_SHARED((N_PAD,), jnp.float32),
            pltpu.SemaphoreType.DMA,
        ],
    )(dst2d, ones_hbm, zeros_hbm)


STAGE = 80


def _spmm_body(table_per_core, src_hbm, dst_hbm, h_hbm, out_hbm,
               srcv, dstv, rows_v, acc_sh, sem):
    c = lax.axis_index("c")
    s = lax.axis_index("s")
    if table_per_core:
        table = h_hbm.at[c]
        nrows = EROWS // NS
        row0 = s * nrows
    else:
        table = h_hbm
        nrows = EROWS // (NC * NS)
        row0 = (c * NS + s) * nrows

    tile_rows = pl.ds(s * ROWS_PER_TILE, ROWS_PER_TILE)
    pltpu.sync_copy(table.at[tile_rows], acc_sh.at[tile_rows])
    plsc.subcore_barrier()

    def step(g, carry):
        pltpu.async_copy(table.at[srcv.at[g]], rows_v, sem).wait()
        pltpu.sync_copy(rows_v, acc_sh.at[dstv.at[g]], add=True)
        return carry

    for st in range(nrows // STAGE):
        pltpu.sync_copy(src_hbm.at[pl.ds(row0 + st * STAGE, STAGE)], srcv)
        pltpu.sync_copy(dst_hbm.at[pl.ds(row0 + st * STAGE, STAGE)], dstv)
        lax.fori_loop(0, STAGE, step, 0, unroll=False)
    plsc.subcore_barrier()
    pltpu.sync_copy(acc_sh.at[tile_rows], out_hbm.at[c].at[tile_rows])


def _spmm(src2d, dst2d, table, table_per_core):
    return pl.kernel(
        functools.partial(_spmm_body, table_per_core),
        out_type=jax.ShapeDtypeStruct((NC, N_PAD, F), jnp.float32),
        mesh=_mesh(),
        scratch_types=[
            pltpu.VMEM((STAGE, 128), jnp.int32),
            pltpu.VMEM((STAGE, 128), jnp.int32),
            pltpu.VMEM((128, F), jnp.float32),
            pltpu.VMEM_SHARED((N_PAD, F), jnp.float32),
            pltpu.SemaphoreType.DMA,
        ],
    )(src2d, dst2d, table)


BN = 256


def _k1_body(x_ref, w_ref, deg_ref, out_ref):
    dinv = lax.rsqrt(deg_ref[0, :] + deg_ref[1, :] + 1.0)
    t = jnp.dot(x_ref[...], w_ref[...], preferred_element_type=jnp.float32)
    out_ref[0] = t * dinv[:, None]


def _k1(xp, W1, deg2):
    return pl.pallas_call(
        _k1_body,
        grid=(NC, N_PAD // BN),
        in_specs=[
            pl.BlockSpec((BN, 128), lambda c, i: (i, 0)),
            pl.BlockSpec((128, F), lambda c, i: (0, c)),
            pl.BlockSpec((NC, BN), lambda c, i: (0, i)),
        ],
        out_specs=pl.BlockSpec((1, BN, F), lambda c, i: (c, i, 0)),
        out_shape=jax.ShapeDtypeStruct((NC, N_PAD, F), jnp.float32),
    )(xp, W1, deg2)


def _k2_body(s1_ref, deg_ref, b1_ref, w2_ref, out_ref):
    dinv = lax.rsqrt(deg_ref[0, :] + deg_ref[1, :] + 1.0)[:, None]
    b1 = b1_ref[...]
    z0 = jnp.maximum(s1_ref[0] * dinv + b1[:F][None, :], 0.0)
    z1 = jnp.maximum(s1_ref[1] * dinv + b1[F:][None, :], 0.0)
    t = (jnp.dot(z0, w2_ref[:F], preferred_element_type=jnp.float32)
         + jnp.dot(z1, w2_ref[F:], preferred_element_type=jnp.float32))
    out_ref[...] = t * dinv


def _k2(s1, deg2, b1, W2):
    return pl.pallas_call(
        _k2_body,
        grid=(N_PAD // BN,),
        in_specs=[
            pl.BlockSpec((NC, BN, F), lambda i: (0, i, 0)),
            pl.BlockSpec((NC, BN), lambda i: (0, i)),
            pl.BlockSpec((2 * F,), lambda i: (0,)),
            pl.BlockSpec((2 * F, F), lambda i: (0, 0)),
        ],
        out_specs=pl.BlockSpec((BN, F), lambda i: (i, 0)),
        out_shape=jax.ShapeDtypeStruct((N_PAD, F), jnp.float32),
    )(s1, deg2, b1, W2)


def _k3_body(s2_ref, h2p_ref, deg_ref, b2_ref, out_ref):
    dinv = lax.rsqrt(deg_ref[0, :] + deg_ref[1, :] + 1.0)[:, None]
    agg = s2_ref[0] + s2_ref[1] - h2p_ref[...]
    out_ref[...] = agg * dinv + b2_ref[...][None, :]


def _k3(s2, h2p, deg2, b2):
    return pl.pallas_call(
        _k3_body,
        grid=(N_PAD // BN,),
        in_specs=[
            pl.BlockSpec((NC, BN, F), lambda i: (0, i, 0)),
            pl.BlockSpec((BN, F), lambda i: (i, 0)),
            pl.BlockSpec((NC, BN), lambda i: (0, i)),
            pl.BlockSpec((F,), lambda i: (0,)),
        ],
        out_specs=pl.BlockSpec((BN, F), lambda i: (i, 0)),
        out_shape=jax.ShapeDtypeStruct((N_PAD, F), jnp.float32),
    )(s2, h2p, deg2, b2)


def kernel(x, edge_index, W1, b1, W2, b2):
    e = edge_index.astype(jnp.int32)
    src, dst = e[0], e[1]
    pad_n = E_PAD - src.shape[0]
    pad = jnp.arange(pad_n, dtype=jnp.int32)
    src_p = jnp.concatenate([src, pad % 61])
    dst_p = jnp.concatenate([dst, N + (pad % 64)])
    src2d = src_p.reshape(EROWS, 128)
    dst2d = dst_p.reshape(EROWS, 128)
    xp = jnp.zeros((N_PAD, 128), jnp.float32).at[:N].set(x)
    ones_h = jnp.ones((128,), jnp.float32)
    zeros_h = jnp.zeros((ROWS_PER_TILE,), jnp.float32)

    deg2 = _deg_kernel(dst2d, ones_h, zeros_h)
    h1p = _k1(xp, W1, deg2)
    s1 = _spmm(src2d, dst2d, h1p, table_per_core=True)
    h2p = _k2(s1, deg2, b1, W2)
    s2 = _spmm(src2d, dst2d, h2p, table_per_core=False)
    out = _k3(s2, h2p, deg2, b2)
    return out[:N]


# speedup vs baseline: 20.5662x; 1.2375x over previous
import functools

import jax
import jax.numpy as jnp
from jax import lax
from jax.experimental import pallas as pl
from jax.experimental.pallas import tpu as pltpu
from jax.experimental.pallas import tpu_sc as plsc

N = 10000
N_PAD = 10240
F = 128
E_PAD = 327680
EROWS = E_PAD // 128
NC, NS = 2, 16
ROWS_PER_TILE = N_PAD // NS


def _mesh():
    return plsc.VectorSubcoreMesh(core_axis_name="c", subcore_axis_name="s",
                                  num_cores=NC, num_subcores=NS)


def _deg_body(dst_hbm, ones_hbm, zeros_hbm, deg_out, dstv, onesv, zerosv,
              deg_sh, sem):
    c = lax.axis_index("c")
    s = lax.axis_index("s")
    nrows = EROWS // (NC * NS)
    row0 = (c * NS + s) * nrows

    pltpu.sync_copy(zeros_hbm.at[pl.ds(0, ROWS_PER_TILE)], zerosv)
    pltpu.sync_copy(zerosv, deg_sh.at[pl.ds(s * ROWS_PER_TILE, ROWS_PER_TILE)])
    pltpu.sync_copy(ones_hbm.at[pl.ds(0, 128)], onesv)
    pltpu.sync_copy(dst_hbm.at[pl.ds(row0, nrows)], dstv)
    plsc.subcore_barrier()

    def step(g, carry):
        pltpu.sync_copy(onesv, deg_sh.at[dstv.at[g]], add=True)
        return carry

    lax.fori_loop(0, nrows, step, 0, unroll=False)
    plsc.subcore_barrier()
    pltpu.sync_copy(deg_sh.at[pl.ds(s * ROWS_PER_TILE, ROWS_PER_TILE)],
                    deg_out.at[c, pl.ds(s * ROWS_PER_TILE, ROWS_PER_TILE)])


def _deg_kernel(dst2d, ones_hbm, zeros_hbm):
    return pl.kernel(
        _deg_body,
        out_type=jax.ShapeDtypeStruct((NC, N_PAD), jnp.float32),
        mesh=_mesh(),
        scratch_types=[
            pltpu.VMEM((EROWS // (NC * NS), 128), jnp.int32),
            pltpu.VMEM((128,), jnp.float32),
            pltpu.VMEM((ROWS_PER_TILE,), jnp.float32),
            pltpu.VMEM_SHARED((N_PAD,), jnp.float32),
            pltpu.SemaphoreType.DMA,
        ],
    )(dst2d, ones_hbm, zeros_hbm)


STAGE = 40


def _spmm_body(table_per_core, src_hbm, dst_hbm, h_hbm, out_hbm,
               srcv, dstv, rows_a, rows_b, acc_sh, sem_a, sem_b):
    c = lax.axis_index("c")
    s = lax.axis_index("s")
    if table_per_core:
        table = h_hbm.at[c]
        nrows = EROWS // NS
        row0 = s * nrows
    else:
        table = h_hbm
        nrows = EROWS // (NC * NS)
        row0 = (c * NS + s) * nrows

    tile_rows = pl.ds(s * ROWS_PER_TILE, ROWS_PER_TILE)
    pltpu.sync_copy(table.at[tile_rows], acc_sh.at[tile_rows])
    plsc.subcore_barrier()

    def pair(p, carry):
        g0 = 2 * p
        pltpu.make_async_copy(table.at[srcv.at[g0]], rows_a, sem_a).wait()
        pltpu.async_copy(table.at[srcv.at[g0 + 1]], rows_b, sem_b)
        pltpu.sync_copy(rows_a, acc_sh.at[dstv.at[g0]], add=True)
        pltpu.make_async_copy(table.at[srcv.at[g0 + 1]], rows_b, sem_b).wait()
        g2 = jnp.where(g0 + 2 < STAGE, g0 + 2, 0)
        pltpu.async_copy(table.at[srcv.at[g2]], rows_a, sem_a)
        pltpu.sync_copy(rows_b, acc_sh.at[dstv.at[g0 + 1]], add=True)
        return carry

    for st in range(nrows // STAGE):
        pltpu.sync_copy(src_hbm.at[pl.ds(row0 + st * STAGE, STAGE)], srcv)
        pltpu.sync_copy(dst_hbm.at[pl.ds(row0 + st * STAGE, STAGE)], dstv)
        pltpu.async_copy(table.at[srcv.at[0]], rows_a, sem_a)
        lax.fori_loop(0, STAGE // 2, pair, 0, unroll=False)
        pltpu.make_async_copy(table.at[srcv.at[0]], rows_a, sem_a).wait()
    plsc.subcore_barrier()
    pltpu.sync_copy(acc_sh.at[tile_rows], out_hbm.at[c].at[tile_rows])


def _spmm(src2d, dst2d, table, table_per_core):
    return pl.kernel(
        functools.partial(_spmm_body, table_per_core),
        out_type=jax.ShapeDtypeStruct((NC, N_PAD, F), jnp.float32),
        mesh=_mesh(),
        scratch_types=[
            pltpu.VMEM((STAGE, 128), jnp.int32),
            pltpu.VMEM((STAGE, 128), jnp.int32),
            pltpu.VMEM((128, F), jnp.float32),
            pltpu.VMEM((128, F), jnp.float32),
            pltpu.VMEM_SHARED((N_PAD, F), jnp.float32),
            pltpu.SemaphoreType.DMA,
            pltpu.SemaphoreType.DMA,
        ],
    )(src2d, dst2d, table)


BN = 256


def _k1_body(x_ref, w_ref, deg_ref, out_ref):
    dinv = lax.rsqrt(deg_ref[0, :] + deg_ref[1, :] + 1.0)
    t = jnp.dot(x_ref[...], w_ref[...], preferred_element_type=jnp.float32)
    out_ref[0] = t * dinv[:, None]


def _k1(xp, W1, deg2):
    return pl.pallas_call(
        _k1_body,
        grid=(NC, N_PAD // BN),
        in_specs=[
            pl.BlockSpec((BN, 128), lambda c, i: (i, 0)),
            pl.BlockSpec((128, F), lambda c, i: (0, c)),
            pl.BlockSpec((NC, BN), lambda c, i: (0, i)),
        ],
        out_specs=pl.BlockSpec((1, BN, F), lambda c, i: (c, i, 0)),
        out_shape=jax.ShapeDtypeStruct((NC, N_PAD, F), jnp.float32),
    )(xp, W1, deg2)


def _k2_body(s1_ref, deg_ref, b1_ref, w2_ref, out_ref):
    dinv = lax.rsqrt(deg_ref[0, :] + deg_ref[1, :] + 1.0)[:, None]
    b1 = b1_ref[...]
    z0 = jnp.maximum(s1_ref[0] * dinv + b1[:F][None, :], 0.0)
    z1 = jnp.maximum(s1_ref[1] * dinv + b1[F:][None, :], 0.0)
    t = (jnp.dot(z0, w2_ref[:F], preferred_element_type=jnp.float32)
         + jnp.dot(z1, w2_ref[F:], preferred_element_type=jnp.float32))
    out_ref[...] = t * dinv


def _k2(s1, deg2, b1, W2):
    return pl.pallas_call(
        _k2_body,
        grid=(N_PAD // BN,),
        in_specs=[
            pl.BlockSpec((NC, BN, F), lambda i: (0, i, 0)),
            pl.BlockSpec((NC, BN), lambda i: (0, i)),
            pl.BlockSpec((2 * F,), lambda i: (0,)),
            pl.BlockSpec((2 * F, F), lambda i: (0, 0)),
        ],
        out_specs=pl.BlockSpec((BN, F), lambda i: (i, 0)),
        out_shape=jax.ShapeDtypeStruct((N_PAD, F), jnp.float32),
    )(s1, deg2, b1, W2)


def _k3_body(s2_ref, h2p_ref, deg_ref, b2_ref, out_ref):
    dinv = lax.rsqrt(deg_ref[0, :] + deg_ref[1, :] + 1.0)[:, None]
    agg = s2_ref[0] + s2_ref[1] - h2p_ref[...]
    out_ref[...] = agg * dinv + b2_ref[...][None, :]


def _k3(s2, h2p, deg2, b2):
    return pl.pallas_call(
        _k3_body,
        grid=(N_PAD // BN,),
        in_specs=[
            pl.BlockSpec((NC, BN, F), lambda i: (0, i, 0)),
            pl.BlockSpec((BN, F), lambda i: (i, 0)),
            pl.BlockSpec((NC, BN), lambda i: (0, i)),
            pl.BlockSpec((F,), lambda i: (0,)),
        ],
        out_specs=pl.BlockSpec((BN, F), lambda i: (i, 0)),
        out_shape=jax.ShapeDtypeStruct((N_PAD, F), jnp.float32),
    )(s2, h2p, deg2, b2)


def kernel(x, edge_index, W1, b1, W2, b2):
    e = edge_index.astype(jnp.int32)
    src, dst = e[0], e[1]
    pad_n = E_PAD - src.shape[0]
    pad = jnp.arange(pad_n, dtype=jnp.int32)
    src_p = jnp.concatenate([src, pad % 61])
    dst_p = jnp.concatenate([dst, N + (pad % 64)])
    src2d = src_p.reshape(EROWS, 128)
    dst2d = dst_p.reshape(EROWS, 128)
    xp = jnp.zeros((N_PAD, 128), jnp.float32).at[:N].set(x)
    ones_h = jnp.ones((128,), jnp.float32)
    zeros_h = jnp.zeros((ROWS_PER_TILE,), jnp.float32)

    deg2 = _deg_kernel(dst2d, ones_h, zeros_h)
    h1p = _k1(xp, W1, deg2)
    s1 = _spmm(src2d, dst2d, h1p, table_per_core=True)
    h2p = _k2(s1, deg2, b1, W2)
    s2 = _spmm(src2d, dst2d, h2p, table_per_core=False)
    out = _k3(s2, h2p, deg2, b2)
    return out[:N]


# speedup vs baseline: 23.1691x; 1.1266x over previous
import functools

import jax
import jax.numpy as jnp
from jax import lax
from jax.experimental import pallas as pl
from jax.experimental.pallas import tpu as pltpu
from jax.experimental.pallas import tpu_sc as plsc

N = 10000
N_PAD = 10240
F = 128
E_PAD = 327680
EROWS = E_PAD // 128
NC, NS = 2, 16
ROWS_PER_TILE = N_PAD // NS


def _mesh():
    return plsc.VectorSubcoreMesh(core_axis_name="c", subcore_axis_name="s",
                                  num_cores=NC, num_subcores=NS)


def _deg_body(dst_hbm, ones_hbm, zeros_hbm, deg_out, dstv, onesv, zerosv,
              deg_sh, sem):
    c = lax.axis_index("c")
    s = lax.axis_index("s")
    nrows = EROWS // (NC * NS)
    row0 = (c * NS + s) * nrows

    pltpu.sync_copy(zeros_hbm.at[pl.ds(0, ROWS_PER_TILE)], zerosv)
    pltpu.sync_copy(zerosv, deg_sh.at[pl.ds(s * ROWS_PER_TILE, ROWS_PER_TILE)])
    pltpu.sync_copy(ones_hbm.at[pl.ds(0, 128)], onesv)
    pltpu.sync_copy(dst_hbm.at[pl.ds(row0, nrows)], dstv)
    pltpu.sync_copy(deg_sh.at[pl.ds(s * ROWS_PER_TILE + ROWS_PER_TILE - 128,
                                    128)], onesv)
    pltpu.sync_copy(ones_hbm.at[pl.ds(0, 128)], onesv)
    plsc.subcore_barrier()

    def step(g, carry):
        pltpu.sync_copy(onesv, deg_sh.at[dstv.at[g]], add=True)
        return carry

    lax.fori_loop(0, nrows, step, 0, unroll=False)
    plsc.subcore_barrier()
    pltpu.sync_copy(dst_hbm.at[pl.ds(row0, 8)], dstv.at[pl.ds(0, 8)])
    plsc.subcore_barrier()
    pltpu.sync_copy(deg_sh.at[pl.ds(s * ROWS_PER_TILE, ROWS_PER_TILE)],
                    deg_out.at[c, pl.ds(s * ROWS_PER_TILE, ROWS_PER_TILE)])


def _deg_kernel(dst2d, ones_hbm, zeros_hbm):
    return pl.kernel(
        _deg_body,
        out_type=jax.ShapeDtypeStruct((NC, N_PAD), jnp.float32),
        mesh=_mesh(),
        scratch_types=[
            pltpu.VMEM((EROWS // (NC * NS), 128), jnp.int32),
            pltpu.VMEM((128,), jnp.float32),
            pltpu.VMEM((ROWS_PER_TILE,), jnp.float32),
            pltpu.VMEM_SHARED((N_PAD,), jnp.float32),
            pltpu.SemaphoreType.DMA,
        ],
    )(dst2d, ones_hbm, zeros_hbm)


STAGE = 40


def _spmm_body(table_per_core, src_hbm, dst_hbm, h_hbm, out_hbm,
               srcv, dstv, rows_a, rows_b, acc_sh, gsem_a, gsem_b):
    c = lax.axis_index("c")
    s = lax.axis_index("s")
    if table_per_core:
        table = h_hbm.at[c]
        nrows = EROWS // NS
        row0 = s * nrows
    else:
        table = h_hbm
        nrows = EROWS // (NC * NS)
        row0 = (c * NS + s) * nrows

    tile_rows = pl.ds(s * ROWS_PER_TILE, ROWS_PER_TILE)
    pltpu.sync_copy(table.at[tile_rows], acc_sh.at[tile_rows])
    pltpu.sync_copy(
        acc_sh.at[pl.ds(s * ROWS_PER_TILE + ROWS_PER_TILE - 128, 128)],
        rows_a)
    plsc.subcore_barrier()

    def pair(p, carry):
        g0 = 2 * p
        pltpu.make_async_copy(table.at[srcv.at[g0]], rows_a, gsem_a).wait()
        pltpu.async_copy(table.at[srcv.at[g0 + 1]], rows_b, gsem_b)
        pltpu.sync_copy(rows_a, acc_sh.at[dstv.at[g0]], add=True)
        pltpu.make_async_copy(table.at[srcv.at[g0 + 1]], rows_b, gsem_b).wait()
        g2 = jnp.where(g0 + 2 < STAGE, g0 + 2, 0)
        pltpu.async_copy(table.at[srcv.at[g2]], rows_a, gsem_a)
        pltpu.sync_copy(rows_b, acc_sh.at[dstv.at[g0 + 1]], add=True)
        return carry

    for st in range(nrows // STAGE):
        pltpu.sync_copy(src_hbm.at[pl.ds(row0 + st * STAGE, STAGE)], srcv)
        pltpu.sync_copy(dst_hbm.at[pl.ds(row0 + st * STAGE, STAGE)], dstv)
        pltpu.async_copy(table.at[srcv.at[0]], rows_a, gsem_a)
        lax.fori_loop(0, STAGE // 2, pair, 0, unroll=False)
        pltpu.make_async_copy(table.at[srcv.at[0]], rows_a, gsem_a).wait()
    plsc.subcore_barrier()
    pltpu.sync_copy(table.at[pl.ds(0, 128)], rows_a)
    plsc.subcore_barrier()
    pltpu.sync_copy(acc_sh.at[tile_rows], out_hbm.at[c].at[tile_rows])


def _spmm(src2d, dst2d, table, table_per_core):
    return pl.kernel(
        functools.partial(_spmm_body, table_per_core),
        out_type=jax.ShapeDtypeStruct((NC, N_PAD, F), jnp.float32),
        mesh=_mesh(),
        scratch_types=[
            pltpu.VMEM((STAGE, 128), jnp.int32),
            pltpu.VMEM((STAGE, 128), jnp.int32),
            pltpu.VMEM((128, F), jnp.float32),
            pltpu.VMEM((128, F), jnp.float32),
            pltpu.VMEM_SHARED((N_PAD, F), jnp.float32),
            pltpu.SemaphoreType.DMA,
            pltpu.SemaphoreType.DMA,
        ],
    )(src2d, dst2d, table)


BN = 512
BN3 = 1024


def _k1_body(x_ref, w_ref, deg_ref, out_ref):
    dinv = lax.rsqrt(deg_ref[0, :] + deg_ref[1, :] + 1.0)[:, None]
    t = jnp.dot(x_ref[...], w_ref[...], preferred_element_type=jnp.float32)
    out_ref[0] = t[:, :F] * dinv
    out_ref[1] = t[:, F:] * dinv


def _k1(xp, W1, deg2):
    return pl.pallas_call(
        _k1_body,
        grid=(N_PAD // BN,),
        in_specs=[
            pl.BlockSpec((BN, 128), lambda i: (i, 0)),
            pl.BlockSpec((128, 2 * F), lambda i: (0, 0)),
            pl.BlockSpec((NC, BN), lambda i: (0, i)),
        ],
        out_specs=pl.BlockSpec((NC, BN, F), lambda i: (0, i, 0)),
        out_shape=jax.ShapeDtypeStruct((NC, N_PAD, F), jnp.float32),
    )(xp, W1, deg2)


def _k2_body(s1_ref, deg_ref, b1_ref, w2_ref, out_ref):
    dinv = lax.rsqrt(deg_ref[0, :] + deg_ref[1, :] + 1.0)[:, None]
    b1 = b1_ref[...]
    z0 = jnp.maximum(s1_ref[0] * dinv + b1[:F][None, :], 0.0)
    z1 = jnp.maximum(s1_ref[1] * dinv + b1[F:][None, :], 0.0)
    t = (jnp.dot(z0, w2_ref[:F], preferred_element_type=jnp.float32)
         + jnp.dot(z1, w2_ref[F:], preferred_element_type=jnp.float32))
    out_ref[...] = t * dinv


def _k2(s1, deg2, b1, W2):
    return pl.pallas_call(
        _k2_body,
        grid=(N_PAD // BN,),
        in_specs=[
            pl.BlockSpec((NC, BN, F), lambda i: (0, i, 0)),
            pl.BlockSpec((NC, BN), lambda i: (0, i)),
            pl.BlockSpec((2 * F,), lambda i: (0,)),
            pl.BlockSpec((2 * F, F), lambda i: (0, 0)),
        ],
        out_specs=pl.BlockSpec((BN, F), lambda i: (i, 0)),
        out_shape=jax.ShapeDtypeStruct((N_PAD, F), jnp.float32),
    )(s1, deg2, b1, W2)


def _k3_body(s2_ref, h2p_ref, deg_ref, b2_ref, out_ref):
    dinv = lax.rsqrt(deg_ref[0, :] + deg_ref[1, :] + 1.0)[:, None]
    agg = s2_ref[0] + s2_ref[1] - h2p_ref[...]
    out_ref[...] = agg * dinv + b2_ref[...][None, :]


def _k3(s2, h2p, deg2, b2):
    return pl.pallas_call(
        _k3_body,
        grid=(N_PAD // BN3,),
        in_specs=[
            pl.BlockSpec((NC, BN3, F), lambda i: (0, i, 0)),
            pl.BlockSpec((BN3, F), lambda i: (i, 0)),
            pl.BlockSpec((NC, BN3), lambda i: (0, i)),
            pl.BlockSpec((F,), lambda i: (0,)),
        ],
        out_specs=pl.BlockSpec((BN3, F), lambda i: (i, 0)),
        out_shape=jax.ShapeDtypeStruct((N, F), jnp.float32),
    )(s2, h2p, deg2, b2)


def kernel(x, edge_index, W1, b1, W2, b2):
    e = edge_index.astype(jnp.int32)
    ne = e.shape[1]
    pad_n = E_PAD - ne
    pad = jnp.arange(pad_n, dtype=jnp.int32)
    pad2d = jnp.stack([pad % 61, N + (pad % 64)]).reshape(2, pad_n // 128, 128)
    e3 = jnp.concatenate([e.reshape(2, ne // 128, 128), pad2d], axis=1)
    src2d = e3[0]
    dst2d = e3[1]
    xp = jnp.zeros((N_PAD, 128), jnp.float32).at[:N].set(x)
    ones_h = jnp.ones((128,), jnp.float32)
    zeros_h = jnp.zeros((ROWS_PER_TILE,), jnp.float32)

    deg2 = _deg_kernel(dst2d, ones_h, zeros_h)
    h1p = _k1(xp, W1, deg2)
    s1 = _spmm(src2d, dst2d, h1p, table_per_core=True)
    h2p = _k2(s1, deg2, b1, W2)
    s2 = _spmm(src2d, dst2d, h2p, table_per_core=False)
    return _k3(s2, h2p, deg2, b2)
